# Initial kernel scaffold; baseline (speedup 1.0000x reference)
import jax
import jax.numpy as jnp
from jax import lax
from jax.experimental import pallas as pl
from jax.experimental.pallas import tpu as pltpu
from jax.experimental.pallas import tpu_sc as plsc

N = 10000
NPAD = 10240
E = 160000
D = 256
DH = 128
K = 8
NTILES = 16
ROWS_PER_TILE = NPAD // NTILES
SUB = 64
ECHUNK = 100
EDGES_PER_TILE = E // NTILES
NCHUNK = EDGES_PER_TILE // ECHUNK
DEG_EDGES_PER_TILE = E // (2 * NTILES)
DEG_NCHUNK = DEG_EDGES_PER_TILE // ECHUNK
ROWBLK = 512
NBLK = NPAD // ROWBLK

_mesh = plsc.VectorSubcoreMesh(core_axis_name="c", subcore_axis_name="s")
_f32 = jnp.float32


def _deg_body(dst_hbm, ones_hbm, zeros16_hbm, dega_hbm, degb_hbm,
              acc, idx_v, ones_v):
    c = lax.axis_index("c")
    s = lax.axis_index("s")
    r0 = s * ROWS_PER_TILE
    pltpu.sync_copy(zeros16_hbm, acc.at[pl.ds(r0, ROWS_PER_TILE)])
    pltpu.sync_copy(ones_hbm, ones_v)
    plsc.subcore_barrier()
    base = (c * NTILES + s) * DEG_EDGES_PER_TILE

    def estep(i, carry):
        e0 = base + i * ECHUNK
        pltpu.sync_copy(dst_hbm.at[pl.ds(e0, ECHUNK)], idx_v)
        pltpu.sync_copy(ones_v, acc.at[idx_v], add=True)
        return carry

    lax.fori_loop(0, DEG_NCHUNK, estep, 0)
    plsc.subcore_barrier()

    @pl.when(c == 0)
    def _():
        pltpu.sync_copy(acc.at[pl.ds(r0, ROWS_PER_TILE)],
                        dega_hbm.at[pl.ds(r0, ROWS_PER_TILE)])

    @pl.when(c == 1)
    def _():
        pltpu.sync_copy(acc.at[pl.ds(r0, ROWS_PER_TILE)],
                        degb_hbm.at[pl.ds(r0, ROWS_PER_TILE)])


_deg_kernel = pl.kernel(
    _deg_body,
    out_type=(jax.ShapeDtypeStruct((NPAD, 16), _f32),
              jax.ShapeDtypeStruct((NPAD, 16), _f32)),
    mesh=_mesh,
    scratch_types=[
        pltpu.VMEM_SHARED((NPAD, 16), _f32),
        pltpu.VMEM((ECHUNK,), jnp.int32),
        pltpu.VMEM((ECHUNK, 16), _f32),
    ],
)


def _make_step(first):
    def body(*refs):
        if first:
            (g_lo, g_hi, src_hbm, dst_hbm, norm16_hbm, zeros_hbm,
             xk_lo, xk_hi, gk_lo, gk_hi,
             acc, idx_v, dst_v, rows_v, accv, nrmv, xkv, gkv, gsem) = refs
            xpp_lo = xpp_hi = xppv = None
        else:
            (g_lo, g_hi, xpp_lo, xpp_hi, src_hbm, dst_hbm, norm16_hbm,
             zeros_hbm, xk_lo, xk_hi, gk_lo, gk_hi,
             acc, idx_v, dst_v, rows_v, accv, nrmv, xkv, gkv, xppv,
             gsem) = refs
        c = lax.axis_index("c")
        s = lax.axis_index("s")

        def half(g_hbm, xpp_hbm, xkh_hbm, gkh_hbm):
            r0t = s * ROWS_PER_TILE
            pltpu.sync_copy(zeros_hbm, acc.at[pl.ds(r0t, ROWS_PER_TILE)])
            plsc.subcore_barrier()
            ebase = s * EDGES_PER_TILE

            def estep(i, carry):
                e0 = ebase + i * ECHUNK
                pltpu.sync_copy(src_hbm.at[pl.ds(e0, ECHUNK)], idx_v)
                pltpu.sync_copy(dst_hbm.at[pl.ds(e0, ECHUNK)], dst_v)
                pltpu.async_copy(g_hbm.at[idx_v], rows_v, gsem).wait()
                pltpu.sync_copy(rows_v, acc.at[dst_v], add=True)
                return carry

            lax.fori_loop(0, NCHUNK, estep, 0)
            plsc.subcore_barrier()

            def nstep(j, carry):
                r0 = r0t + j * SUB
                pltpu.sync_copy(acc.at[pl.ds(r0, SUB)], accv)
                pltpu.sync_copy(norm16_hbm.at[pl.ds(r0, SUB)], nrmv)
                if not first:
                    pltpu.sync_copy(xpp_hbm.at[pl.ds(r0, SUB)], xppv)

                def rstep(n, carry2):
                    sp = nrmv[n]
                    for jj in range(DH // 16):
                        a = accv[n, pl.ds(jj * 16, 16)]
                        if first:
                            xk = -(sp * a)
                        else:
                            xk = ((-2.0) * sp * a
                                  - xppv[n, pl.ds(jj * 16, 16)])
                        xkv[n, pl.ds(jj * 16, 16)] = xk
                        gkv[n, pl.ds(jj * 16, 16)] = xk * sp
                    return carry2

                lax.fori_loop(0, SUB, rstep, 0)
                pltpu.sync_copy(xkv, xkh_hbm.at[pl.ds(r0, SUB)])
                pltpu.sync_copy(gkv, gkh_hbm.at[pl.ds(r0, SUB)])
                return carry

            lax.fori_loop(0, ROWS_PER_TILE // SUB, nstep, 0)

        @pl.when(c == 0)
        def _():
            half(g_lo, xpp_lo, xk_lo, gk_lo)

        @pl.when(c == 1)
        def _():
            half(g_hi, xpp_hi, xk_hi, gk_hi)

    scratch = [
        pltpu.VMEM_SHARED((NPAD, DH), _f32),
        pltpu.VMEM((ECHUNK,), jnp.int32),
        pltpu.VMEM((ECHUNK,), jnp.int32),
        pltpu.VMEM((ECHUNK, DH), _f32),
        pltpu.VMEM((SUB, DH), _f32),
        pltpu.VMEM((SUB, 16), _f32),
        pltpu.VMEM((SUB, DH), _f32),
        pltpu.VMEM((SUB, DH), _f32),
    ]
    if not first:
        scratch.append(pltpu.VMEM((SUB, DH), _f32))
    scratch.append(pltpu.SemaphoreType.DMA)
    return pl.kernel(
        body,
        out_type=tuple(jax.ShapeDtypeStruct((NPAD, DH), _f32)
                       for _ in range(4)),
        mesh=_mesh,
        scratch_types=scratch,
    )


_step_first = _make_step(True)
_step_rest = _make_step(False)


def _prologue_body(dega, degb, xlo, xhi, n16, glo, ghi):
    deg = dega[...] + degb[...]
    nrm = jax.lax.rsqrt(jnp.maximum(deg, 1.0))
    n16[...] = nrm
    nl = nrm[:, :1]
    glo[...] = xlo[...] * nl
    ghi[...] = xhi[...] * nl


def _prologue(dega, degb, xlo, xhi):
    blk16 = pl.BlockSpec((ROWBLK, 16), lambda i: (i, 0))
    blkd = pl.BlockSpec((ROWBLK, DH), lambda i: (i, 0))
    return pl.pallas_call(
        _prologue_body,
        grid=(NBLK,),
        in_specs=[blk16, blk16, blkd, blkd],
        out_specs=[blk16, blkd, blkd],
        out_shape=(jax.ShapeDtypeStruct((NPAD, 16), _f32),
                   jax.ShapeDtypeStruct((NPAD, DH), _f32),
                   jax.ShapeDtypeStruct((NPAD, DH), _f32)),
    )(dega, degb, xlo, xhi)


def _make_matmul(with_relu):
    def body(*refs):
        if with_relu:
            xs = refs[:16]
            w_ref, b_ref, n16_ref = refs[16:19]
            hlo_o, hhi_o, glo_o, ghi_o = refs[19:]
        else:
            xs = refs[:16]
            w_ref, b_ref = refs[16:18]
            (out_o,) = refs[18:]
        acc = None
        for k in range(K):
            xl = xs[2 * k][...]
            xh = xs[2 * k + 1][...]
            p_lo = jnp.dot(xl, w_ref[2 * k * DH:(2 * k + 1) * DH, :],
                           preferred_element_type=_f32)
            p_hi = jnp.dot(xh, w_ref[(2 * k + 1) * DH:(2 * k + 2) * DH, :],
                           preferred_element_type=_f32)
            contrib = p_lo + p_hi
            acc = contrib if acc is None else acc + contrib
        acc = acc + b_ref[...]
        if with_relu:
            h = jnp.maximum(acc, 0.0)
            nl = n16_ref[...][:, :1]
            hlo_o[...] = h[:, :DH]
            hhi_o[...] = h[:, DH:]
            glo_o[...] = h[:, :DH] * nl
            ghi_o[...] = h[:, DH:] * nl
        else:
            out_o[...] = acc

    blkd = pl.BlockSpec((ROWBLK, DH), lambda i: (i, 0))
    blkw = pl.BlockSpec((K * D, D), lambda i: (0, 0))
    blkb = pl.BlockSpec((1, D), lambda i: (0, 0))
    blk16 = pl.BlockSpec((ROWBLK, 16), lambda i: (i, 0))
    blkfull = pl.BlockSpec((ROWBLK, D), lambda i: (i, 0))
    in_specs = [blkd] * 16 + [blkw, blkb]
    if with_relu:
        in_specs = in_specs + [blk16]
        out_specs = [blkd, blkd, blkd, blkd]
        out_shape = tuple(jax.ShapeDtypeStruct((NPAD, DH), _f32)
                          for _ in range(4))
    else:
        out_specs = [blkfull]
        out_shape = (jax.ShapeDtypeStruct((NPAD, D), _f32),)

    def call(xs, w, b, n16=None):
        args = list(xs) + [w, b]
        if with_relu:
            args.append(n16)
        return pl.pallas_call(
            body,
            grid=(NBLK,),
            in_specs=in_specs,
            out_specs=out_specs,
            out_shape=out_shape,
        )(*args)

    return call


_matmul_relu = _make_matmul(True)
_matmul_final = _make_matmul(False)


def _cheb_layer(x_pair, g_pair, src, dst, norm16, zeros_d, w, b, relu, n16):
    xs = [x_pair[0], x_pair[1]]
    xk_lo, xk_hi, gk_lo, gk_hi = _step_first(
        g_pair[0], g_pair[1], src, dst, norm16, zeros_d)
    xs += [xk_lo, xk_hi]
    prev2 = x_pair
    prev = (xk_lo, xk_hi)
    g = (gk_lo, gk_hi)
    for _ in range(2, K):
        xk_lo, xk_hi, gk_lo, gk_hi = _step_rest(
            g[0], g[1], prev2[0], prev2[1], src, dst, norm16, zeros_d)
        xs += [xk_lo, xk_hi]
        prev2 = prev
        prev = (xk_lo, xk_hi)
        g = (gk_lo, gk_hi)
    if relu:
        return _matmul_relu(xs, w, b, n16)
    return _matmul_final(xs, w, b)[0]


def kernel(in_feat, edge_index, W1, b1, W2, b2):
    src = edge_index[0]
    dst = edge_index[1]
    xp = jnp.pad(in_feat, ((0, NPAD - N), (0, 0)))
    x_lo = xp[:, :DH]
    x_hi = xp[:, DH:]
    ones_e = jnp.ones((ECHUNK, 16), _f32)
    zeros16 = jnp.zeros((ROWS_PER_TILE, 16), _f32)
    zeros_d = jnp.zeros((ROWS_PER_TILE, DH), _f32)
    b1r = b1.reshape(1, D)
    b2r = b2.reshape(1, D)

    dega, degb = _deg_kernel(dst, ones_e, zeros16)
    norm16, g0_lo, g0_hi = _prologue(dega, degb, x_lo, x_hi)

    h_lo, h_hi, g1_lo, g1_hi = _cheb_layer(
        (x_lo, x_hi), (g0_lo, g0_hi), src, dst, norm16, zeros_d,
        W1, b1r, True, norm16)
    out = _cheb_layer(
        (h_lo, h_hi), (g1_lo, g1_hi), src, dst, norm16, zeros_d,
        W2, b2r, False, None)
    return out[:N]


# speedup vs baseline: 2.8743x; 2.8743x over previous
import jax
import jax.numpy as jnp
from jax import lax
from jax.experimental import pallas as pl
from jax.experimental.pallas import tpu as pltpu
from jax.experimental.pallas import tpu_sc as plsc

N = 10000
NPAD = 10240
E = 160000
D = 256
DH = 128
K = 8
NTILES = 16
ROWS_PER_TILE = NPAD // NTILES
SUB = 64
ECHUNK = 80
EDGES_PER_TILE = E // NTILES
NCHUNK = EDGES_PER_TILE // ECHUNK
ROWBLK = 512
NBLK = NPAD // ROWBLK

_mesh = plsc.VectorSubcoreMesh(core_axis_name="c", subcore_axis_name="s")
_f32 = jnp.float32


def _deg_body(dst_hbm, ones_hbm, zeros_hbm, deg_hbm, acc, idx_v, ones_v):
    c = lax.axis_index("c")
    s = lax.axis_index("s")

    @pl.when(c == 0)
    def _():
        r0 = s * ROWS_PER_TILE
        pltpu.sync_copy(zeros_hbm, acc.at[pl.ds(r0, ROWS_PER_TILE)])
        pltpu.sync_copy(ones_hbm, ones_v)
        plsc.subcore_barrier()
        base = s * EDGES_PER_TILE

        def estep(i, carry):
            e0 = base + i * ECHUNK
            pltpu.sync_copy(dst_hbm.at[pl.ds(e0, ECHUNK)], idx_v.at[0])
            pltpu.sync_copy(ones_v, acc.at[idx_v.at[0]], add=True)
            return carry

        lax.fori_loop(0, NCHUNK, estep, 0)
        plsc.subcore_barrier()
        pltpu.sync_copy(acc.at[pl.ds(r0, ROWS_PER_TILE)],
                        deg_hbm.at[pl.ds(r0, ROWS_PER_TILE)])


_deg_kernel = pl.kernel(
    _deg_body,
    out_type=jax.ShapeDtypeStruct((NPAD, DH), _f32),
    mesh=_mesh,
    scratch_types=[
        pltpu.VMEM_SHARED((NPAD, DH), _f32),
        pltpu.VMEM((1, ECHUNK), jnp.int32),
        pltpu.VMEM((ECHUNK, DH), _f32),
    ],
)


def _make_step(first):
    def body(*refs):
        if first:
            (g_lo, g_hi, src_hbm, dst_hbm, norm16_hbm, zeros_hbm,
             xk_lo, xk_hi, gk_lo, gk_hi,
             acc, idx_v, dst_v, rows_v, accv, nrmv, xkv, xgv, gsem) = refs
            xpp_lo = xpp_hi = None
        else:
            (g_lo, g_hi, xpp_lo, xpp_hi, src_hbm, dst_hbm, norm16_hbm,
             zeros_hbm, xk_lo, xk_hi, gk_lo, gk_hi,
             acc, idx_v, dst_v, rows_v, accv, nrmv, xkv, xgv,
             gsem) = refs
        c = lax.axis_index("c")
        s = lax.axis_index("s")

        def half(g_hbm, xpp_hbm, xkh_hbm, gkh_hbm):
            r0t = s * ROWS_PER_TILE
            pltpu.sync_copy(zeros_hbm, acc.at[pl.ds(r0t, ROWS_PER_TILE)])
            plsc.subcore_barrier()
            ebase = s * EDGES_PER_TILE

            def estep(i, carry):
                e0 = ebase + i * ECHUNK
                pltpu.sync_copy(src_hbm.at[pl.ds(e0, ECHUNK)], idx_v.at[0])
                pltpu.sync_copy(dst_hbm.at[pl.ds(e0, ECHUNK)], dst_v.at[0])
                pltpu.async_copy(g_hbm.at[idx_v.at[0]], rows_v, gsem).wait()
                pltpu.sync_copy(rows_v, acc.at[dst_v.at[0]], add=True)
                return carry

            lax.fori_loop(0, NCHUNK, estep, 0)
            plsc.subcore_barrier()

            def nstep(j, carry):
                r0 = r0t + j * SUB
                pltpu.sync_copy(acc.at[pl.ds(r0, SUB)], accv)
                pltpu.sync_copy(norm16_hbm.at[pl.ds(r0, SUB)], nrmv)
                if not first:
                    pltpu.sync_copy(xpp_hbm.at[pl.ds(r0, SUB)], xgv)

                def rstep(n, carry2):
                    sp = nrmv[n]
                    for jj in range(DH // 16):
                        a = accv[n, pl.ds(jj * 16, 16)]
                        if first:
                            xk = -(sp * a)
                        else:
                            xk = ((-2.0) * sp * a
                                  - xgv[n, pl.ds(jj * 16, 16)])
                        xkv[n, pl.ds(jj * 16, 16)] = xk
                        xgv[n, pl.ds(jj * 16, 16)] = xk * sp
                    return carry2

                lax.fori_loop(0, SUB, rstep, 0)
                pltpu.sync_copy(xkv, xkh_hbm.at[pl.ds(r0, SUB)])
                pltpu.sync_copy(xgv, gkh_hbm.at[pl.ds(r0, SUB)])
                return carry

            lax.fori_loop(0, ROWS_PER_TILE // SUB, nstep, 0)

        @pl.when(c == 0)
        def _():
            half(g_lo, xpp_lo, xk_lo, gk_lo)

        @pl.when(c == 1)
        def _():
            half(g_hi, xpp_hi, xk_hi, gk_hi)

    scratch = [
        pltpu.VMEM_SHARED((NPAD, DH), _f32),
        pltpu.VMEM((1, ECHUNK), jnp.int32),
        pltpu.VMEM((1, ECHUNK), jnp.int32),
        pltpu.VMEM((ECHUNK, DH), _f32),
        pltpu.VMEM((SUB, DH), _f32),
        pltpu.VMEM((SUB, 16), _f32),
        pltpu.VMEM((SUB, DH), _f32),
        pltpu.VMEM((SUB, DH), _f32),
        pltpu.SemaphoreType.DMA,
    ]
    return pl.kernel(
        body,
        out_type=tuple(jax.ShapeDtypeStruct((NPAD, DH), _f32)
                       for _ in range(4)),
        mesh=_mesh,
        scratch_types=scratch,
    )


_step_first = _make_step(True)
_step_rest = _make_step(False)


def _prologue_body(dega, xlo, xhi, n16, glo, ghi):
    deg = dega[...][:, :16]
    nrm = jax.lax.rsqrt(jnp.maximum(deg, 1.0))
    n16[...] = nrm
    nl = nrm[:, :1]
    glo[...] = xlo[...] * nl
    ghi[...] = xhi[...] * nl


def _prologue(dega, xlo, xhi):
    blk16 = pl.BlockSpec((ROWBLK, 16), lambda i: (i, 0))
    blkd = pl.BlockSpec((ROWBLK, DH), lambda i: (i, 0))
    return pl.pallas_call(
        _prologue_body,
        grid=(NBLK,),
        in_specs=[blkd, blkd, blkd],
        out_specs=[blk16, blkd, blkd],
        out_shape=(jax.ShapeDtypeStruct((NPAD, 16), _f32),
                   jax.ShapeDtypeStruct((NPAD, DH), _f32),
                   jax.ShapeDtypeStruct((NPAD, DH), _f32)),
    )(dega, xlo, xhi)


def _make_matmul(with_relu):
    def body(*refs):
        if with_relu:
            xs = refs[:16]
            w_ref, b_ref, n16_ref = refs[16:19]
            hlo_o, hhi_o, glo_o, ghi_o = refs[19:]
        else:
            xs = refs[:16]
            w_ref, b_ref = refs[16:18]
            (out_o,) = refs[18:]
        acc = None
        for k in range(K):
            xl = xs[2 * k][...]
            xh = xs[2 * k + 1][...]
            p_lo = jnp.dot(xl, w_ref[2 * k * DH:(2 * k + 1) * DH, :],
                           preferred_element_type=_f32)
            p_hi = jnp.dot(xh, w_ref[(2 * k + 1) * DH:(2 * k + 2) * DH, :],
                           preferred_element_type=_f32)
            contrib = p_lo + p_hi
            acc = contrib if acc is None else acc + contrib
        acc = acc + b_ref[...]
        if with_relu:
            h = jnp.maximum(acc, 0.0)
            nl = n16_ref[...][:, :1]
            hlo_o[...] = h[:, :DH]
            hhi_o[...] = h[:, DH:]
            glo_o[...] = h[:, :DH] * nl
            ghi_o[...] = h[:, DH:] * nl
        else:
            out_o[...] = acc

    blkd = pl.BlockSpec((ROWBLK, DH), lambda i: (i, 0))
    blkw = pl.BlockSpec((K * D, D), lambda i: (0, 0))
    blkb = pl.BlockSpec((1, D), lambda i: (0, 0))
    blk16 = pl.BlockSpec((ROWBLK, 16), lambda i: (i, 0))
    blkfull = pl.BlockSpec((ROWBLK, D), lambda i: (i, 0))
    in_specs = [blkd] * 16 + [blkw, blkb]
    if with_relu:
        in_specs = in_specs + [blk16]
        out_specs = [blkd, blkd, blkd, blkd]
        out_shape = tuple(jax.ShapeDtypeStruct((NPAD, DH), _f32)
                          for _ in range(4))
    else:
        out_specs = [blkfull]
        out_shape = (jax.ShapeDtypeStruct((NPAD, D), _f32),)

    def call(xs, w, b, n16=None):
        args = list(xs) + [w, b]
        if with_relu:
            args.append(n16)
        return pl.pallas_call(
            body,
            grid=(NBLK,),
            in_specs=in_specs,
            out_specs=out_specs,
            out_shape=out_shape,
        )(*args)

    return call


_matmul_relu = _make_matmul(True)
_matmul_final = _make_matmul(False)


def _cheb_layer(x_pair, g_pair, src, dst, norm16, zeros_d, w, b, relu, n16):
    step1 = _step_first
    stepn = _step_rest
    xs = [x_pair[0], x_pair[1]]
    xk_lo, xk_hi, gk_lo, gk_hi = step1(
        g_pair[0], g_pair[1], src, dst, norm16, zeros_d)
    xs += [xk_lo, xk_hi]
    prev2 = x_pair
    prev = (xk_lo, xk_hi)
    g = (gk_lo, gk_hi)
    for _ in range(2, K):
        xk_lo, xk_hi, gk_lo, gk_hi = stepn(
            g[0], g[1], prev2[0], prev2[1], src, dst, norm16, zeros_d)
        xs += [xk_lo, xk_hi]
        prev2 = prev
        prev = (xk_lo, xk_hi)
        g = (gk_lo, gk_hi)
    if relu:
        return _matmul_relu(xs, w, b, n16)
    return _matmul_final(xs, w, b)[0]


def kernel(in_feat, edge_index, W1, b1, W2, b2):
    src = edge_index[0]
    dst = edge_index[1]
    xp = jnp.pad(in_feat, ((0, NPAD - N), (0, 0)))
    x_lo = xp[:, :DH]
    x_hi = xp[:, DH:]
    ones_e = jnp.ones((ECHUNK, DH), _f32)
    zeros_d = jnp.zeros((ROWS_PER_TILE, DH), _f32)
    b1r = b1.reshape(1, D)
    b2r = b2.reshape(1, D)

    dega = _deg_kernel(dst, ones_e, zeros_d)
    norm16, g0_lo, g0_hi = _prologue(dega, x_lo, x_hi)

    h_lo, h_hi, g1_lo, g1_hi = _cheb_layer(
        (x_lo, x_hi), (g0_lo, g0_hi), src, dst, norm16, zeros_d,
        W1, b1r, True, norm16)
    out = _cheb_layer(
        (h_lo, h_hi), (g1_lo, g1_hi), src, dst, norm16, zeros_d,
        W2, b2r, False, None)
    return out[:N]


# speedup vs baseline: 4.4197x; 1.5377x over previous
import jax
import jax.numpy as jnp
from jax import lax
from jax.experimental import pallas as pl
from jax.experimental.pallas import tpu as pltpu
from jax.experimental.pallas import tpu_sc as plsc

N = 10000
NPAD = 10240
E = 160000
D = 256
DH = 128
K = 8
NTILES = 16
ROWS_PER_TILE = NPAD // NTILES
SUB = 64
ECHUNK = 80
EDGES_PER_TILE = E // NTILES
NCHUNK = EDGES_PER_TILE // ECHUNK
ROWBLK = 512
NBLK = NPAD // ROWBLK

_mesh = plsc.VectorSubcoreMesh(core_axis_name="c", subcore_axis_name="s")
_f32 = jnp.float32


def _deg_body(dst_hbm, ones_hbm, zeros_hbm, deg_hbm, acc, idx_v, ones_v):
    c = lax.axis_index("c")
    s = lax.axis_index("s")

    @pl.when(c == 0)
    def _():
        r0 = s * ROWS_PER_TILE
        pltpu.sync_copy(zeros_hbm, acc.at[pl.ds(r0, ROWS_PER_TILE)])
        pltpu.sync_copy(ones_hbm, ones_v)
        plsc.subcore_barrier()
        base = s * EDGES_PER_TILE

        def estep(i, carry):
            e0 = base + i * ECHUNK
            pltpu.sync_copy(dst_hbm.at[pl.ds(e0, ECHUNK)], idx_v.at[0])
            pltpu.sync_copy(ones_v, acc.at[idx_v.at[0]], add=True)
            return carry

        lax.fori_loop(0, NCHUNK, estep, 0)
        plsc.subcore_barrier()
        pltpu.sync_copy(acc.at[pl.ds(r0, ROWS_PER_TILE)],
                        deg_hbm.at[pl.ds(r0, ROWS_PER_TILE)])


_deg_kernel = pl.kernel(
    _deg_body,
    out_type=jax.ShapeDtypeStruct((NPAD, DH), _f32),
    mesh=_mesh,
    scratch_types=[
        pltpu.VMEM_SHARED((NPAD, DH), _f32),
        pltpu.VMEM((1, ECHUNK), jnp.int32),
        pltpu.VMEM((ECHUNK, DH), _f32),
    ],
)


def _make_step(first):
    def body(*refs):
        if first:
            (g_lo, g_hi, src_hbm, dst_hbm, norm16_hbm, zeros_hbm,
             xk_lo, xk_hi, gk_lo, gk_hi,
             acc, idx0, idx1, dst0, dst1, rows0, rows1,
             nrmv, xgv, gsem0, gsem1) = refs
            xpp_lo = xpp_hi = None
        else:
            (g_lo, g_hi, xpp_lo, xpp_hi, src_hbm, dst_hbm, norm16_hbm,
             zeros_hbm, xk_lo, xk_hi, gk_lo, gk_hi,
             acc, idx0, idx1, dst0, dst1, rows0, rows1,
             nrmv, xgv, gsem0, gsem1) = refs
        c = lax.axis_index("c")
        s = lax.axis_index("s")

        def half(g_hbm, xpp_hbm, xkh_hbm, gkh_hbm):
            r0t = s * ROWS_PER_TILE
            pltpu.sync_copy(zeros_hbm, acc.at[pl.ds(r0t, ROWS_PER_TILE)])
            plsc.subcore_barrier()
            ebase = s * EDGES_PER_TILE
            bufs = ((idx0, dst0, rows0, gsem0), (idx1, dst1, rows1, gsem1))

            def fetch(cid, b):
                ib, db, rb, sb = bufs[b]
                e0 = ebase + cid * ECHUNK
                pltpu.sync_copy(src_hbm.at[pl.ds(e0, ECHUNK)], ib.at[0])
                pltpu.sync_copy(dst_hbm.at[pl.ds(e0, ECHUNK)], db.at[0])
                pltpu.async_copy(g_hbm.at[ib.at[0]], rb, sb)

            def consume(b):
                ib, db, rb, sb = bufs[b]
                pltpu.make_async_copy(g_hbm.at[ib.at[0]], rb, sb).wait()
                pltpu.sync_copy(rb, acc.at[db.at[0]], add=True)

            fetch(0, 0)
            fetch(1, 1)

            def estep(i, carry):
                for b in range(2):
                    cid = 2 * i + b
                    consume(b)

                    @pl.when(cid + 2 < NCHUNK)
                    def _():
                        fetch(cid + 2, b)
                return carry

            lax.fori_loop(0, NCHUNK // 2, estep, 0)
            consume(0)
            plsc.subcore_barrier()

            def nstep(j, carry):
                r0 = r0t + j * SUB
                pltpu.sync_copy(acc.at[pl.ds(r0, SUB)],
                                rows0.at[pl.ds(0, SUB)])
                pltpu.sync_copy(norm16_hbm.at[pl.ds(r0, SUB)], nrmv)
                if not first:
                    pltpu.sync_copy(xpp_hbm.at[pl.ds(r0, SUB)], xgv)

                def rstep(n, carry2):
                    sp = nrmv[n]
                    for jj in range(DH // 16):
                        a = rows0[n, pl.ds(jj * 16, 16)]
                        if first:
                            xk = -(sp * a)
                        else:
                            xk = ((-2.0) * sp * a
                                  - xgv[n, pl.ds(jj * 16, 16)])
                        rows1[n, pl.ds(jj * 16, 16)] = xk
                        xgv[n, pl.ds(jj * 16, 16)] = xk * sp
                    return carry2

                lax.fori_loop(0, SUB, rstep, 0)
                pltpu.sync_copy(rows1.at[pl.ds(0, SUB)],
                                xkh_hbm.at[pl.ds(r0, SUB)])
                pltpu.sync_copy(xgv, gkh_hbm.at[pl.ds(r0, SUB)])
                return carry

            lax.fori_loop(0, ROWS_PER_TILE // SUB, nstep, 0)

        @pl.when(c == 0)
        def _():
            half(g_lo, xpp_lo, xk_lo, gk_lo)

        @pl.when(c == 1)
        def _():
            half(g_hi, xpp_hi, xk_hi, gk_hi)

    scratch = [
        pltpu.VMEM_SHARED((NPAD, DH), _f32),
        pltpu.VMEM((1, ECHUNK), jnp.int32),
        pltpu.VMEM((1, ECHUNK), jnp.int32),
        pltpu.VMEM((1, ECHUNK), jnp.int32),
        pltpu.VMEM((1, ECHUNK), jnp.int32),
        pltpu.VMEM((ECHUNK, DH), _f32),
        pltpu.VMEM((ECHUNK, DH), _f32),
        pltpu.VMEM((SUB, 16), _f32),
        pltpu.VMEM((SUB, DH), _f32),
        pltpu.SemaphoreType.DMA,
        pltpu.SemaphoreType.DMA,
    ]
    return pl.kernel(
        body,
        out_type=tuple(jax.ShapeDtypeStruct((NPAD, DH), _f32)
                       for _ in range(4)),
        mesh=_mesh,
        scratch_types=scratch,
    )


_step_first = _make_step(True)
_step_rest = _make_step(False)


def _prologue_body(dega, xlo, xhi, n16, glo, ghi):
    deg = dega[...][:, :16]
    nrm = jax.lax.rsqrt(jnp.maximum(deg, 1.0))
    n16[...] = nrm
    nl = nrm[:, :1]
    glo[...] = xlo[...] * nl
    ghi[...] = xhi[...] * nl


def _prologue(dega, xlo, xhi):
    blk16 = pl.BlockSpec((ROWBLK, 16), lambda i: (i, 0))
    blkd = pl.BlockSpec((ROWBLK, DH), lambda i: (i, 0))
    return pl.pallas_call(
        _prologue_body,
        grid=(NBLK,),
        in_specs=[blkd, blkd, blkd],
        out_specs=[blk16, blkd, blkd],
        out_shape=(jax.ShapeDtypeStruct((NPAD, 16), _f32),
                   jax.ShapeDtypeStruct((NPAD, DH), _f32),
                   jax.ShapeDtypeStruct((NPAD, DH), _f32)),
    )(dega, xlo, xhi)


def _make_matmul(with_relu):
    def body(*refs):
        if with_relu:
            xs = refs[:16]
            w_ref, b_ref, n16_ref = refs[16:19]
            hlo_o, hhi_o, glo_o, ghi_o = refs[19:]
        else:
            xs = refs[:16]
            w_ref, b_ref = refs[16:18]
            (out_o,) = refs[18:]
        acc = None
        for k in range(K):
            xl = xs[2 * k][...]
            xh = xs[2 * k + 1][...]
            p_lo = jnp.dot(xl, w_ref[2 * k * DH:(2 * k + 1) * DH, :],
                           preferred_element_type=_f32)
            p_hi = jnp.dot(xh, w_ref[(2 * k + 1) * DH:(2 * k + 2) * DH, :],
                           preferred_element_type=_f32)
            contrib = p_lo + p_hi
            acc = contrib if acc is None else acc + contrib
        acc = acc + b_ref[...]
        if with_relu:
            h = jnp.maximum(acc, 0.0)
            nl = n16_ref[...][:, :1]
            hlo_o[...] = h[:, :DH]
            hhi_o[...] = h[:, DH:]
            glo_o[...] = h[:, :DH] * nl
            ghi_o[...] = h[:, DH:] * nl
        else:
            out_o[...] = acc

    blkd = pl.BlockSpec((ROWBLK, DH), lambda i: (i, 0))
    blkw = pl.BlockSpec((K * D, D), lambda i: (0, 0))
    blkb = pl.BlockSpec((1, D), lambda i: (0, 0))
    blk16 = pl.BlockSpec((ROWBLK, 16), lambda i: (i, 0))
    blkfull = pl.BlockSpec((ROWBLK, D), lambda i: (i, 0))
    in_specs = [blkd] * 16 + [blkw, blkb]
    if with_relu:
        in_specs = in_specs + [blk16]
        out_specs = [blkd, blkd, blkd, blkd]
        out_shape = tuple(jax.ShapeDtypeStruct((NPAD, DH), _f32)
                          for _ in range(4))
    else:
        out_specs = [blkfull]
        out_shape = (jax.ShapeDtypeStruct((NPAD, D), _f32),)

    def call(xs, w, b, n16=None):
        args = list(xs) + [w, b]
        if with_relu:
            args.append(n16)
        return pl.pallas_call(
            body,
            grid=(NBLK,),
            in_specs=in_specs,
            out_specs=out_specs,
            out_shape=out_shape,
        )(*args)

    return call


_matmul_relu = _make_matmul(True)
_matmul_final = _make_matmul(False)


def _cheb_layer(x_pair, g_pair, src, dst, norm16, zeros_d, w, b, relu, n16):
    step1 = _step_first
    stepn = _step_rest
    xs = [x_pair[0], x_pair[1]]
    xk_lo, xk_hi, gk_lo, gk_hi = step1(
        g_pair[0], g_pair[1], src, dst, norm16, zeros_d)
    xs += [xk_lo, xk_hi]
    prev2 = x_pair
    prev = (xk_lo, xk_hi)
    g = (gk_lo, gk_hi)
    for _ in range(2, K):
        xk_lo, xk_hi, gk_lo, gk_hi = stepn(
            g[0], g[1], prev2[0], prev2[1], src, dst, norm16, zeros_d)
        xs += [xk_lo, xk_hi]
        prev2 = prev
        prev = (xk_lo, xk_hi)
        g = (gk_lo, gk_hi)
    if relu:
        return _matmul_relu(xs, w, b, n16)
    return _matmul_final(xs, w, b)[0]


def kernel(in_feat, edge_index, W1, b1, W2, b2):
    src = edge_index[0]
    dst = edge_index[1]
    xp = jnp.pad(in_feat, ((0, NPAD - N), (0, 0)))
    x_lo = xp[:, :DH]
    x_hi = xp[:, DH:]
    ones_e = jnp.ones((ECHUNK, DH), _f32)
    zeros_d = jnp.zeros((ROWS_PER_TILE, DH), _f32)
    b1r = b1.reshape(1, D)
    b2r = b2.reshape(1, D)

    dega = _deg_kernel(dst, ones_e, zeros_d)
    norm16, g0_lo, g0_hi = _prologue(dega, x_lo, x_hi)

    h_lo, h_hi, g1_lo, g1_hi = _cheb_layer(
        (x_lo, x_hi), (g0_lo, g0_hi), src, dst, norm16, zeros_d,
        W1, b1r, True, norm16)
    out = _cheb_layer(
        (h_lo, h_hi), (g1_lo, g1_hi), src, dst, norm16, zeros_d,
        W2, b2r, False, None)
    return out[:N]


# speedup vs baseline: 5.0693x; 1.1470x over previous
import jax
import jax.numpy as jnp
from jax import lax
from jax.experimental import pallas as pl
from jax.experimental.pallas import tpu as pltpu
from jax.experimental.pallas import tpu_sc as plsc

N = 10000
NPAD = 10240
E = 160000
D = 256
DH = 128
K = 8
NTILES = 16
ROWS_PER_TILE = NPAD // NTILES
SUB = 64
ECHUNK = 80
EDGES_PER_TILE = E // NTILES
NCHUNK = EDGES_PER_TILE // ECHUNK
ROWBLK = 512
NBLK = NPAD // ROWBLK

_mesh = plsc.VectorSubcoreMesh(core_axis_name="c", subcore_axis_name="s")
_f32 = jnp.float32


def _deg_body(pairs_hbm, ones_hbm, zeros_hbm, deg_hbm,
              acc, ip0, ip1, ones_v, is0, is1):
    c = lax.axis_index("c")
    s = lax.axis_index("s")

    @pl.when(c == 0)
    def _():
        r0 = s * ROWS_PER_TILE
        pltpu.sync_copy(zeros_hbm, acc.at[pl.ds(r0, ROWS_PER_TILE)])
        pltpu.sync_copy(ones_hbm, ones_v)
        plsc.subcore_barrier()
        bufs = ((ip0, is0), (ip1, is1))

        def idx_start(cid, b):
            ib, isem = bufs[b]
            pltpu.async_copy(pairs_hbm.at[s].at[cid], ib, isem)

        def do_chunk(cid, b, dynamic):
            ib, isem = bufs[b]
            pltpu.make_async_copy(pairs_hbm.at[s].at[cid], ib, isem).wait()
            pltpu.sync_copy(ones_v, acc.at[ib.at[1]], add=True)

            def prefetch():
                idx_start(cid + 2, b)

            if dynamic:
                pl.when(cid + 2 < NCHUNK)(prefetch)
            elif cid + 2 < NCHUNK:
                prefetch()

        idx_start(0, 0)
        idx_start(1, 1)

        def estep(i, carry):
            for b in range(2):
                do_chunk(2 * i + b, b, True)
            return carry

        lax.fori_loop(0, NCHUNK // 2, estep, 0)
        for cid in range(2 * (NCHUNK // 2), NCHUNK):
            do_chunk(cid, cid % 2, False)
        plsc.subcore_barrier()
        pltpu.sync_copy(acc.at[pl.ds(r0, ROWS_PER_TILE)],
                        deg_hbm.at[pl.ds(r0, ROWS_PER_TILE)])


_deg_kernel = pl.kernel(
    _deg_body,
    out_type=jax.ShapeDtypeStruct((NPAD, DH), _f32),
    mesh=_mesh,
    scratch_types=[
        pltpu.VMEM_SHARED((NPAD, DH), _f32),
        pltpu.VMEM((2, ECHUNK), jnp.int32),
        pltpu.VMEM((2, ECHUNK), jnp.int32),
        pltpu.VMEM((ECHUNK, DH), _f32),
        pltpu.SemaphoreType.DMA,
        pltpu.SemaphoreType.DMA,
    ],
)


def _make_step(first):
    def body(*refs):
        if first:
            (g_lo, g_hi, pairs_hbm, norm16_hbm, zeros_hbm,
             xk_lo, xk_hi, gk_lo, gk_hi,
             acc, ip0, ip1, ip2, rows0, rows1, rows2,
             nrmv, xgv, is0, is1, is2, gs0, gs1, gs2) = refs
            xpp_lo = xpp_hi = None
        else:
            (g_lo, g_hi, xpp_lo, xpp_hi, pairs_hbm, norm16_hbm,
             zeros_hbm, xk_lo, xk_hi, gk_lo, gk_hi,
             acc, ip0, ip1, ip2, rows0, rows1, rows2,
             nrmv, xgv, is0, is1, is2, gs0, gs1, gs2) = refs
        c = lax.axis_index("c")
        s = lax.axis_index("s")

        def half(g_hbm, xpp_hbm, xkh_hbm, gkh_hbm):
            r0t = s * ROWS_PER_TILE
            pltpu.sync_copy(zeros_hbm, acc.at[pl.ds(r0t, ROWS_PER_TILE)])
            plsc.subcore_barrier()
            bufs = ((ip0, rows0, is0, gs0), (ip1, rows1, is1, gs1),
                    (ip2, rows2, is2, gs2))

            def idx_start(cid, b):
                ib, rb, isem, gsem = bufs[b]
                pltpu.async_copy(pairs_hbm.at[s].at[cid], ib, isem)

            def idx_wait(cid, b):
                ib, rb, isem, gsem = bufs[b]
                pltpu.make_async_copy(pairs_hbm.at[s].at[cid], ib,
                                      isem).wait()

            def gather_start(b):
                ib, rb, isem, gsem = bufs[b]
                pltpu.async_copy(g_hbm.at[ib.at[0]], rb, gsem)

            def gather_wait(b):
                ib, rb, isem, gsem = bufs[b]
                pltpu.make_async_copy(g_hbm.at[ib.at[0]], rb, gsem).wait()

            def scatter(b):
                ib, rb, isem, gsem = bufs[b]
                pltpu.sync_copy(rb, acc.at[ib.at[1]], add=True)

            def do_chunk(cid, b, dynamic):
                gather_wait(b)
                bn = (b + 1) % 3

                def start_next():
                    idx_wait(cid + 1, bn)
                    gather_start(bn)

                if dynamic or cid + 1 < NCHUNK:
                    if dynamic:
                        pl.when(cid + 1 < NCHUNK)(start_next)
                    else:
                        start_next()
                scatter(b)

                def prefetch():
                    idx_start(cid + 3, b)

                if dynamic or cid + 3 < NCHUNK:
                    if dynamic:
                        pl.when(cid + 3 < NCHUNK)(prefetch)
                    else:
                        prefetch()

            idx_start(0, 0)
            idx_start(1, 1)
            idx_start(2, 2)
            idx_wait(0, 0)
            gather_start(0)

            def estep(i, carry):
                for b in range(3):
                    do_chunk(3 * i + b, b, True)
                return carry

            lax.fori_loop(0, NCHUNK // 3, estep, 0)
            for cid in range(3 * (NCHUNK // 3), NCHUNK):
                do_chunk(cid, cid % 3, False)
            plsc.subcore_barrier()

            def nstep(j, carry):
                r0 = r0t + j * SUB
                pltpu.sync_copy(acc.at[pl.ds(r0, SUB)],
                                rows0.at[pl.ds(0, SUB)])
                pltpu.sync_copy(norm16_hbm.at[pl.ds(r0, SUB)], nrmv)
                if not first:
                    pltpu.sync_copy(xpp_hbm.at[pl.ds(r0, SUB)], xgv)

                def rstep(n, carry2):
                    sp = nrmv[n]
                    for jj in range(DH // 16):
                        a = rows0[n, pl.ds(jj * 16, 16)]
                        if first:
                            xk = -(sp * a)
                        else:
                            xk = ((-2.0) * sp * a
                                  - xgv[n, pl.ds(jj * 16, 16)])
                        rows1[n, pl.ds(jj * 16, 16)] = xk
                        xgv[n, pl.ds(jj * 16, 16)] = xk * sp
                    return carry2

                lax.fori_loop(0, SUB, rstep, 0)
                pltpu.sync_copy(rows1.at[pl.ds(0, SUB)],
                                xkh_hbm.at[pl.ds(r0, SUB)])
                pltpu.sync_copy(xgv, gkh_hbm.at[pl.ds(r0, SUB)])
                return carry

            lax.fori_loop(0, ROWS_PER_TILE // SUB, nstep, 0)

        @pl.when(c == 0)
        def _():
            half(g_lo, xpp_lo, xk_lo, gk_lo)

        @pl.when(c == 1)
        def _():
            half(g_hi, xpp_hi, xk_hi, gk_hi)

    scratch = [
        pltpu.VMEM_SHARED((NPAD, DH), _f32),
        pltpu.VMEM((2, ECHUNK), jnp.int32),
        pltpu.VMEM((2, ECHUNK), jnp.int32),
        pltpu.VMEM((2, ECHUNK), jnp.int32),
        pltpu.VMEM((ECHUNK, DH), _f32),
        pltpu.VMEM((ECHUNK, DH), _f32),
        pltpu.VMEM((ECHUNK, DH), _f32),
        pltpu.VMEM((SUB, 16), _f32),
        pltpu.VMEM((SUB, DH), _f32),
        pltpu.SemaphoreType.DMA,
        pltpu.SemaphoreType.DMA,
        pltpu.SemaphoreType.DMA,
        pltpu.SemaphoreType.DMA,
        pltpu.SemaphoreType.DMA,
        pltpu.SemaphoreType.DMA,
    ]
    return pl.kernel(
        body,
        out_type=tuple(jax.ShapeDtypeStruct((NPAD, DH), _f32)
                       for _ in range(4)),
        mesh=_mesh,
        scratch_types=scratch,
    )


_step_first = _make_step(True)
_step_rest = _make_step(False)


def _prologue_body(dega, xlo, xhi, n16, glo, ghi):
    deg = dega[...][:, :16]
    nrm = jax.lax.rsqrt(jnp.maximum(deg, 1.0))
    n16[...] = nrm
    nl = nrm[:, :1]
    glo[...] = xlo[...] * nl
    ghi[...] = xhi[...] * nl


def _prologue(dega, xlo, xhi):
    blk16 = pl.BlockSpec((ROWBLK, 16), lambda i: (i, 0))
    blkd = pl.BlockSpec((ROWBLK, DH), lambda i: (i, 0))
    return pl.pallas_call(
        _prologue_body,
        grid=(NBLK,),
        in_specs=[blkd, blkd, blkd],
        out_specs=[blk16, blkd, blkd],
        out_shape=(jax.ShapeDtypeStruct((NPAD, 16), _f32),
                   jax.ShapeDtypeStruct((NPAD, DH), _f32),
                   jax.ShapeDtypeStruct((NPAD, DH), _f32)),
    )(dega, xlo, xhi)


def _make_matmul(with_relu):
    def body(*refs):
        if with_relu:
            xs = refs[:16]
            w_ref, b_ref, n16_ref = refs[16:19]
            hlo_o, hhi_o, glo_o, ghi_o = refs[19:]
        else:
            xs = refs[:16]
            w_ref, b_ref = refs[16:18]
            (out_o,) = refs[18:]
        acc = None
        for k in range(K):
            xl = xs[2 * k][...]
            xh = xs[2 * k + 1][...]
            p_lo = jnp.dot(xl, w_ref[2 * k * DH:(2 * k + 1) * DH, :],
                           preferred_element_type=_f32)
            p_hi = jnp.dot(xh, w_ref[(2 * k + 1) * DH:(2 * k + 2) * DH, :],
                           preferred_element_type=_f32)
            contrib = p_lo + p_hi
            acc = contrib if acc is None else acc + contrib
        acc = acc + b_ref[...]
        if with_relu:
            h = jnp.maximum(acc, 0.0)
            nl = n16_ref[...][:, :1]
            hlo_o[...] = h[:, :DH]
            hhi_o[...] = h[:, DH:]
            glo_o[...] = h[:, :DH] * nl
            ghi_o[...] = h[:, DH:] * nl
        else:
            out_o[...] = acc

    blkd = pl.BlockSpec((ROWBLK, DH), lambda i: (i, 0))
    blkw = pl.BlockSpec((K * D, D), lambda i: (0, 0))
    blkb = pl.BlockSpec((1, D), lambda i: (0, 0))
    blk16 = pl.BlockSpec((ROWBLK, 16), lambda i: (i, 0))
    blkfull = pl.BlockSpec((ROWBLK, D), lambda i: (i, 0))
    in_specs = [blkd] * 16 + [blkw, blkb]
    if with_relu:
        in_specs = in_specs + [blk16]
        out_specs = [blkd, blkd, blkd, blkd]
        out_shape = tuple(jax.ShapeDtypeStruct((NPAD, DH), _f32)
                          for _ in range(4))
    else:
        out_specs = [blkfull]
        out_shape = (jax.ShapeDtypeStruct((NPAD, D), _f32),)

    def call(xs, w, b, n16=None):
        args = list(xs) + [w, b]
        if with_relu:
            args.append(n16)
        return pl.pallas_call(
            body,
            grid=(NBLK,),
            in_specs=in_specs,
            out_specs=out_specs,
            out_shape=out_shape,
        )(*args)

    return call


_matmul_relu = _make_matmul(True)
_matmul_final = _make_matmul(False)


def _cheb_layer(x_pair, g_pair, pairs, norm16, zeros_d, w, b, relu, n16):
    xs = [x_pair[0], x_pair[1]]
    xk_lo, xk_hi, gk_lo, gk_hi = _step_first(
        g_pair[0], g_pair[1], pairs, norm16, zeros_d)
    xs += [xk_lo, xk_hi]
    prev2 = x_pair
    prev = (xk_lo, xk_hi)
    g = (gk_lo, gk_hi)
    for _ in range(2, K):
        xk_lo, xk_hi, gk_lo, gk_hi = _step_rest(
            g[0], g[1], prev2[0], prev2[1], pairs, norm16, zeros_d)
        xs += [xk_lo, xk_hi]
        prev2 = prev
        prev = (xk_lo, xk_hi)
        g = (gk_lo, gk_hi)
    if relu:
        return _matmul_relu(xs, w, b, n16)
    return _matmul_final(xs, w, b)[0]


def kernel(in_feat, edge_index, W1, b1, W2, b2):
    src = edge_index[0]
    dst = edge_index[1]
    xp = jnp.pad(in_feat, ((0, NPAD - N), (0, 0)))
    x_lo = xp[:, :DH]
    x_hi = xp[:, DH:]
    ones_e = jnp.ones((ECHUNK, DH), _f32)
    zeros_d = jnp.zeros((ROWS_PER_TILE, DH), _f32)
    b1r = b1.reshape(1, D)
    b2r = b2.reshape(1, D)
    src3 = src.reshape(NTILES, NCHUNK, ECHUNK)
    dst3 = dst.reshape(NTILES, NCHUNK, ECHUNK)
    pairs = jnp.stack([src3, dst3], axis=2)

    dega = _deg_kernel(pairs, ones_e, zeros_d)
    norm16, g0_lo, g0_hi = _prologue(dega, x_lo, x_hi)

    h_lo, h_hi, g1_lo, g1_hi = _cheb_layer(
        (x_lo, x_hi), (g0_lo, g0_hi), pairs, norm16, zeros_d,
        W1, b1r, True, norm16)
    out = _cheb_layer(
        (h_lo, h_hi), (g1_lo, g1_hi), pairs, norm16, zeros_d,
        W2, b2r, False, None)
    return out[:N]
